# baseline (speedup 1.0000x reference)
import functools

import jax
import jax.numpy as jnp
from jax import lax
from jax.experimental import pallas as pl
from jax.experimental.pallas import tpu as pltpu
from jax.experimental.pallas import tpu_sc as plsc

VOCAB = 128
DIM = 128
BATCH = 16384
SC_SHARE = BATCH // 2
TC_BLK = 2048


@functools.cache
def _build_sc(n_sc):
    info = plsc.get_sparse_core_info()
    nc, ns = info.num_cores, info.num_subcores
    nw = nc * ns
    b_per_w = n_sc // nw
    chunk = b_per_w // 2

    mesh = plsc.VectorSubcoreMesh(core_axis_name="c", subcore_axis_name="s")

    @functools.partial(
        pl.kernel,
        mesh=mesh,
        out_type=jax.ShapeDtypeStruct((n_sc, DIM), jnp.float32),
        scratch_types=[
            pltpu.VMEM((b_per_w,), jnp.int32),
            pltpu.VMEM((chunk, DIM), jnp.float32),
            pltpu.VMEM((chunk, DIM), jnp.float32),
            pltpu.VMEM_SHARED((VOCAB, DIM), jnp.float32),
            pltpu.SemaphoreType.DMA,
            pltpu.SemaphoreType.DMA,
            pltpu.SemaphoreType.DMA,
            pltpu.SemaphoreType.DMA,
        ],
    )
    def sc_gather(x_hbm, table_hbm, out_hbm, idx_v, rows0, rows1,
                  table_sh, g0, g1, wsem, tsem):
        s = lax.axis_index("s")
        wid = s * nc + lax.axis_index("c")
        base = wid * b_per_w
        tcopy = pltpu.make_async_copy(table_hbm, table_sh, tsem)

        @pl.when(s == 0)
        def _():
            tcopy.start()

        pltpu.sync_copy(x_hbm.at[pl.ds(base, b_per_w)], idx_v)

        @pl.when(s == 0)
        def _():
            tcopy.wait()

        plsc.subcore_barrier()

        gc0 = pltpu.make_async_copy(
            table_sh.at[idx_v.at[pl.ds(0, chunk)]], rows0, g0)
        gc0.start()
        gc1 = pltpu.make_async_copy(
            table_sh.at[idx_v.at[pl.ds(chunk, chunk)]], rows1, g1)
        gc1.start()
        gc0.wait()
        wc0 = pltpu.make_async_copy(rows0, out_hbm.at[pl.ds(base, chunk)], wsem)
        wc0.start()
        gc1.wait()
        wc1 = pltpu.make_async_copy(
            rows1, out_hbm.at[pl.ds(base + chunk, chunk)], wsem)
        wc1.start()
        wc0.wait()
        wc1.wait()

    return sc_gather


def _tc_body(x_ref, table_ref, out_ref):
    idx = x_ref[0, 0, :]
    onehot = (idx[:, None] == lax.broadcasted_iota(
        jnp.int32, (TC_BLK, VOCAB), 1)).astype(jnp.float32)
    out_ref[...] = jnp.dot(onehot, table_ref[...],
                           preferred_element_type=jnp.float32)


@functools.cache
def _build_tc(n_tc):
    nb = n_tc // TC_BLK
    return pl.pallas_call(
        _tc_body,
        grid=(nb,),
        in_specs=[
            pl.BlockSpec((1, 1, TC_BLK), lambda i: (i, 0, 0)),
            pl.BlockSpec((VOCAB, DIM), lambda i: (0, 0)),
        ],
        out_specs=pl.BlockSpec((TC_BLK, DIM), lambda i: (i, 0)),
        out_shape=jax.ShapeDtypeStruct((n_tc, DIM), jnp.float32),
    )


def kernel(x, table):
    x = x.astype(jnp.int32)
    n_tc = BATCH - SC_SHARE
    sc_out = _build_sc(SC_SHARE)(x[:SC_SHARE], table)
    x_tc = x[SC_SHARE:].reshape(n_tc // TC_BLK, 1, TC_BLK)
    tc_out = _build_tc(n_tc)(x_tc, table)
    return jnp.concatenate([sc_out, tc_out], axis=0)


# speedup vs baseline: 1.1796x; 1.1796x over previous
import functools

import jax
import jax.numpy as jnp
from jax import lax
from jax.experimental import pallas as pl
from jax.experimental.pallas import tpu as pltpu
from jax.experimental.pallas import tpu_sc as plsc

VOCAB = 128
DIM = 128
BATCH = 16384


@functools.cache
def _build():
    info = plsc.get_sparse_core_info()
    nc, ns = info.num_cores, info.num_subcores
    nw = nc * ns
    b_per_w = BATCH // nw
    half = b_per_w // 2

    mesh = plsc.VectorSubcoreMesh(core_axis_name="c", subcore_axis_name="s")

    @functools.partial(
        pl.kernel,
        mesh=mesh,
        out_type=jax.ShapeDtypeStruct((BATCH, DIM), jnp.float32),
        scratch_types=[
            pltpu.VMEM((b_per_w,), jnp.int32),
            pltpu.VMEM((half, DIM), jnp.float32),
            pltpu.VMEM((half, DIM), jnp.float32),
            pltpu.VMEM_SHARED((VOCAB, DIM), jnp.float32),
            pltpu.SemaphoreType.DMA,
            pltpu.SemaphoreType.DMA,
            pltpu.SemaphoreType.DMA,
        ],
    )
    def gather_kernel(x_hbm, table_hbm, out_hbm, idx_v, rows0, rows1,
                      table_sh, g0, g1, wsem):
        s = lax.axis_index("s")
        wid = s * nc + lax.axis_index("c")
        base = wid * b_per_w
        tcopy = pltpu.make_async_copy(table_hbm, table_sh, g1)

        @pl.when(s == 0)
        def _():
            tcopy.start()

        pltpu.sync_copy(x_hbm.at[pl.ds(base, b_per_w)], idx_v)

        @pl.when(s == 0)
        def _():
            tcopy.wait()

        plsc.subcore_barrier()

        gc0 = pltpu.make_async_copy(
            table_sh.at[idx_v.at[pl.ds(0, half)]], rows0, g0)
        gc0.start()
        gc1 = pltpu.make_async_copy(
            table_sh.at[idx_v.at[pl.ds(half, half)]], rows1, g1)
        gc1.start()
        gc0.wait()
        wc0 = pltpu.make_async_copy(rows0, out_hbm.at[pl.ds(base, half)], wsem)
        wc0.start()
        gc1.wait()
        wc1 = pltpu.make_async_copy(
            rows1, out_hbm.at[pl.ds(base + half, half)], wsem)
        wc1.start()
        wc0.wait()
        wc1.wait()

    return gather_kernel


def kernel(x, table):
    return _build()(x.astype(jnp.int32), table)


# speedup vs baseline: 1.2199x; 1.0342x over previous
import functools

import jax
import jax.numpy as jnp
from jax import lax
from jax.experimental import pallas as pl
from jax.experimental.pallas import tpu as pltpu
from jax.experimental.pallas import tpu_sc as plsc

VOCAB = 128
DIM = 128
BATCH = 16384


@functools.cache
def _build():
    info = plsc.get_sparse_core_info()
    nc, ns = info.num_cores, info.num_subcores
    nw = nc * ns
    b_per_w = BATCH // nw
    chunk = 64
    nchunk = b_per_w // chunk

    mesh = plsc.VectorSubcoreMesh(core_axis_name="c", subcore_axis_name="s")

    @functools.partial(
        pl.kernel,
        mesh=mesh,
        out_type=jax.ShapeDtypeStruct((BATCH, DIM), jnp.float32),
        scratch_types=[
            pltpu.VMEM((b_per_w,), jnp.int32),
            pltpu.VMEM((chunk, DIM), jnp.float32),
            pltpu.VMEM((chunk, DIM), jnp.float32),
            pltpu.VMEM_SHARED((VOCAB, DIM), jnp.float32),
            pltpu.SemaphoreType.DMA,
            pltpu.SemaphoreType.DMA,
            pltpu.SemaphoreType.DMA,
            pltpu.SemaphoreType.DMA,
            pltpu.SemaphoreType.DMA,
            pltpu.SemaphoreType.DMA,
        ],
    )
    def gather_kernel(x_hbm, table_hbm, out_hbm, idx_v, rows0, rows1,
                      table_sh, g0, g1, w0, w1, isem, tsem):
        s = lax.axis_index("s")
        wid = s * nc + lax.axis_index("c")
        base = wid * b_per_w
        bufs = (rows0, rows1)
        gsems = (g0, g1)
        wsems = (w0, w1)
        tcopy = pltpu.make_async_copy(table_hbm, table_sh, tsem)

        @pl.when(s == 0)
        def _():
            tcopy.start()

        icopy = pltpu.make_async_copy(
            x_hbm.at[pl.ds(base + chunk, b_per_w - chunk)],
            idx_v.at[pl.ds(chunk, b_per_w - chunk)], isem)
        icopy.start()
        pltpu.sync_copy(x_hbm.at[pl.ds(base, chunk)], idx_v.at[pl.ds(0, chunk)])

        @pl.when(s == 0)
        def _():
            tcopy.wait()

        plsc.subcore_barrier()

        gcs = []
        wcs = []
        for k in range(nchunk):
            b = k % 2
            if k == 1:
                icopy.wait()
            if k >= 2:
                wcs[k - 2].wait()
            gc = pltpu.make_async_copy(
                table_sh.at[idx_v.at[pl.ds(k * chunk, chunk)]], bufs[b], gsems[b])
            gc.start()
            gcs.append(gc)
            if k >= 1:
                gcs[k - 1].wait()
                wc = pltpu.make_async_copy(
                    bufs[(k - 1) % 2], out_hbm.at[pl.ds(base + (k - 1) * chunk, chunk)],
                    wsems[(k - 1) % 2])
                wc.start()
                wcs.append(wc)
        gcs[-1].wait()
        wc = pltpu.make_async_copy(
            bufs[(nchunk - 1) % 2],
            out_hbm.at[pl.ds(base + (nchunk - 1) * chunk, chunk)],
            wsems[(nchunk - 1) % 2])
        wc.start()
        wcs.append(wc)
        wcs[-2].wait()
        wcs[-1].wait()

    return gather_kernel


def kernel(x, table):
    return _build()(x.astype(jnp.int32), table)
